# Initial kernel scaffold; baseline (speedup 1.0000x reference)
import jax
import jax.numpy as jnp
from jax.experimental import pallas as pl


def kernel(x, edge_index, edge_attr, batch, params):
    raise NotImplementedError("write your pallas kernel here")


# speedup vs baseline: 8.0831x; 8.0831x over previous
import functools

import jax
import jax.numpy as jnp
from jax import lax
from jax.experimental import pallas as pl
from jax.experimental.pallas import tpu as pltpu
from jax.experimental.pallas import tpu_sc as plsc

N = 10000
E = 320000
B = 256
IN = 3
HID = 256
NUM_LAYERS = 4
NUM_T = 4

N8 = 10240
BM = 1024
NB = N8 // BM
NTILES = 32
C = 128
EA = 323584
EPT = EA // NTILES
EPT1 = EA // 16
R16 = N8 // 16

f32 = jnp.float32
i32 = jnp.int32


def _leaky(x):
    return jnp.maximum(x, 0.01 * x)


def _elu(x):
    return jnp.where(x > 0, x, jnp.exp(jnp.minimum(x, 0.0)) - 1.0)


def _k0_body(xp, w1p, b1, wx, w2, attr, x1o, uo, v0o, v1o, ro):
    x1 = _leaky(jnp.dot(xp[...], w1p[...]) + b1[...])
    x1o[...] = x1
    uo[...] = jnp.dot(x1, wx[...])
    v = jnp.dot(x1, w2[...])
    v0o[...] = v[:, :128]
    v1o[...] = v[:, 128:]
    ro[...] = jnp.sum(x1 * attr[...], axis=1, keepdims=True)


def _stage0(xp, w1p, b1, wx, w2, attr):
    return pl.pallas_call(
        _k0_body,
        grid=(NB,),
        in_specs=[
            pl.BlockSpec((BM, 128), lambda i: (i, 0)),
            pl.BlockSpec((128, HID), lambda i: (0, 0)),
            pl.BlockSpec((1, HID), lambda i: (0, 0)),
            pl.BlockSpec((HID, HID), lambda i: (0, 0)),
            pl.BlockSpec((HID, HID), lambda i: (0, 0)),
            pl.BlockSpec((1, HID), lambda i: (0, 0)),
        ],
        out_specs=[
            pl.BlockSpec((BM, HID), lambda i: (i, 0)),
            pl.BlockSpec((BM, HID), lambda i: (i, 0)),
            pl.BlockSpec((BM, 128), lambda i: (i, 0)),
            pl.BlockSpec((BM, 128), lambda i: (i, 0)),
            pl.BlockSpec((BM, 1), lambda i: (i, 0)),
        ],
        out_shape=[
            jax.ShapeDtypeStruct((N8, HID), f32),
            jax.ShapeDtypeStruct((N8, HID), f32),
            jax.ShapeDtypeStruct((N8, 128), f32),
            jax.ShapeDtypeStruct((N8, 128), f32),
            jax.ShapeDtypeStruct((N8, 1), f32),
        ],
    )(xp, w1p, b1, wx, w2, attr)


def _gru_body(m, x, bias, wih, whh, bih, bhh, lin, ats, atd,
              xno, hs0o, hs1o, aso, ado):
    h = _elu(jnp.concatenate([m[0], m[1]], axis=-1) + bias[...])
    xv = x[...]
    gi = jnp.dot(h, wih[...]) + bih[...]
    gh = jnp.dot(xv, whh[...]) + bhh[...]
    r = jax.nn.sigmoid(gi[:, :HID] + gh[:, :HID])
    z = jax.nn.sigmoid(gi[:, HID:2 * HID] + gh[:, HID:2 * HID])
    n = jnp.tanh(gi[:, 2 * HID:] + r * gh[:, 2 * HID:])
    xn = jnp.maximum((1 - z) * n + z * xv, 0.0)
    xno[...] = xn
    hs = jnp.dot(xn, lin[...])
    hs0o[...] = hs[:, :128]
    hs1o[...] = hs[:, 128:]
    aso[...] = jnp.sum(hs * ats[...], axis=1, keepdims=True)
    ado[...] = jnp.sum(hs * atd[...], axis=1, keepdims=True)


def _gru_prep(m, x, bias, gp, cp):
    return pl.pallas_call(
        _gru_body,
        grid=(NB,),
        in_specs=[
            pl.BlockSpec((2, BM, 128), lambda i: (0, i, 0)),
            pl.BlockSpec((BM, HID), lambda i: (i, 0)),
            pl.BlockSpec((1, HID), lambda i: (0, 0)),
            pl.BlockSpec((HID, 3 * HID), lambda i: (0, 0)),
            pl.BlockSpec((HID, 3 * HID), lambda i: (0, 0)),
            pl.BlockSpec((1, 3 * HID), lambda i: (0, 0)),
            pl.BlockSpec((1, 3 * HID), lambda i: (0, 0)),
            pl.BlockSpec((HID, HID), lambda i: (0, 0)),
            pl.BlockSpec((1, HID), lambda i: (0, 0)),
            pl.BlockSpec((1, HID), lambda i: (0, 0)),
        ],
        out_specs=[
            pl.BlockSpec((BM, HID), lambda i: (i, 0)),
            pl.BlockSpec((BM, 128), lambda i: (i, 0)),
            pl.BlockSpec((BM, 128), lambda i: (i, 0)),
            pl.BlockSpec((BM, 1), lambda i: (i, 0)),
            pl.BlockSpec((BM, 1), lambda i: (i, 0)),
        ],
        out_shape=[
            jax.ShapeDtypeStruct((N8, HID), f32),
            jax.ShapeDtypeStruct((N8, 128), f32),
            jax.ShapeDtypeStruct((N8, 128), f32),
            jax.ShapeDtypeStruct((N8, 1), f32),
            jax.ShapeDtypeStruct((N8, 1), f32),
        ],
    )(m, x, bias.reshape(1, HID), gp['w_ih'].T, gp['w_hh'].T,
      gp['b_ih'].reshape(1, 3 * HID), gp['b_hh'].reshape(1, 3 * HID),
      cp['lin'].T, cp['att_src'].reshape(1, HID), cp['att_dst'].reshape(1, HID))


def _pool_body(x, batch, lin, atd, outo, adsto):
    i = pl.program_id(0)
    cols = lax.broadcasted_iota(i32, (BM, B), 1)
    mask = (batch[...] == cols).astype(f32)
    contrib = lax.dot_general(mask, x[...], (((0,), (0,)), ((), ())))

    @pl.when(i == 0)
    def _():
        outo[...] = contrib

    @pl.when(i > 0)
    def _():
        outo[...] += contrib

    @pl.when(i == NB - 1)
    def _():
        o = jnp.maximum(outo[...], 0.0)
        outo[...] = o
        hd = jnp.dot(o, lin[...])
        adsto[...] = jnp.sum(hd * atd[...], axis=1, keepdims=True)


def _pool(x, batch_p, cp):
    return pl.pallas_call(
        _pool_body,
        grid=(NB,),
        in_specs=[
            pl.BlockSpec((BM, HID), lambda i: (i, 0)),
            pl.BlockSpec((BM, 1), lambda i: (i, 0)),
            pl.BlockSpec((HID, HID), lambda i: (0, 0)),
            pl.BlockSpec((1, HID), lambda i: (0, 0)),
        ],
        out_specs=[
            pl.BlockSpec((B, HID), lambda i: (0, 0)),
            pl.BlockSpec((B, 1), lambda i: (0, 0)),
        ],
        out_shape=[
            jax.ShapeDtypeStruct((B, HID), f32),
            jax.ShapeDtypeStruct((B, 1), f32),
        ],
    )(x, batch_p, cp['lin'].T, cp['att_dst'].reshape(1, HID))


def _mol_body(hs0, hs1, asrc, batch, adstp, outp,
              wih, whh, bih, bhh, lin, atd, bias, l2, b2,
              outo, adsto, reso, num, den):
    i = pl.program_id(0)
    cols = lax.broadcasted_iota(i32, (BM, B), 1)
    mask = (batch[...] == cols).astype(f32)
    adn = jnp.dot(mask, adstp[...])
    e = jnp.exp(_leaky(asrc[...] + adn))
    hs = jnp.concatenate([hs0[...], hs1[...]], axis=-1)
    he = hs * e
    cn = lax.dot_general(mask, he, (((0,), (0,)), ((), ())))
    cd = lax.dot_general(mask, e, (((0,), (0,)), ((), ())))

    @pl.when(i == 0)
    def _():
        num[...] = cn
        den[...] = cd

    @pl.when(i > 0)
    def _():
        num[...] += cn
        den[...] += cd

    @pl.when(i == NB - 1)
    def _():
        h = _elu(num[...] / (den[...] + 1e-16) + bias[...])
        ov = outp[...]
        gi = jnp.dot(h, wih[...]) + bih[...]
        gh = jnp.dot(ov, whh[...]) + bhh[...]
        r = jax.nn.sigmoid(gi[:, :HID] + gh[:, :HID])
        z = jax.nn.sigmoid(gi[:, HID:2 * HID] + gh[:, HID:2 * HID])
        n = jnp.tanh(gi[:, 2 * HID:] + r * gh[:, 2 * HID:])
        on = jnp.maximum((1 - z) * n + z * ov, 0.0)
        outo[...] = on
        hd = jnp.dot(on, lin[...])
        adsto[...] = jnp.sum(hd * atd[...], axis=1, keepdims=True)
        reso[...] = jnp.dot(on, l2[...]) + b2[...]


def _mol_sweep(hs0, hs1, asrc, batch_p, adstp, outp, gp, cp, l2w, l2b):
    return pl.pallas_call(
        _mol_body,
        grid=(NB,),
        in_specs=[
            pl.BlockSpec((BM, 128), lambda i: (i, 0)),
            pl.BlockSpec((BM, 128), lambda i: (i, 0)),
            pl.BlockSpec((BM, 1), lambda i: (i, 0)),
            pl.BlockSpec((BM, 1), lambda i: (i, 0)),
            pl.BlockSpec((B, 1), lambda i: (0, 0)),
            pl.BlockSpec((B, HID), lambda i: (0, 0)),
            pl.BlockSpec((HID, 3 * HID), lambda i: (0, 0)),
            pl.BlockSpec((HID, 3 * HID), lambda i: (0, 0)),
            pl.BlockSpec((1, 3 * HID), lambda i: (0, 0)),
            pl.BlockSpec((1, 3 * HID), lambda i: (0, 0)),
            pl.BlockSpec((HID, HID), lambda i: (0, 0)),
            pl.BlockSpec((1, HID), lambda i: (0, 0)),
            pl.BlockSpec((1, HID), lambda i: (0, 0)),
            pl.BlockSpec((HID, 32), lambda i: (0, 0)),
            pl.BlockSpec((1, 32), lambda i: (0, 0)),
        ],
        out_specs=[
            pl.BlockSpec((B, HID), lambda i: (0, 0)),
            pl.BlockSpec((B, 1), lambda i: (0, 0)),
            pl.BlockSpec((B, 32), lambda i: (0, 0)),
        ],
        out_shape=[
            jax.ShapeDtypeStruct((B, HID), f32),
            jax.ShapeDtypeStruct((B, 1), f32),
            jax.ShapeDtypeStruct((B, 32), f32),
        ],
        scratch_shapes=[
            pltpu.VMEM((B, HID), f32),
            pltpu.VMEM((B, 1), f32),
        ],
    )(hs0, hs1, asrc, batch_p, adstp, outp,
      gp['w_ih'].T, gp['w_hh'].T, gp['b_ih'].reshape(1, 3 * HID),
      gp['b_hh'].reshape(1, 3 * HID), cp['lin'].T,
      cp['att_dst'].reshape(1, HID), cp['bias'].reshape(1, HID),
      l2w.T, l2b.reshape(1, 32))


_MESH = plsc.VectorSubcoreMesh(core_axis_name="c", subcore_axis_name="s")


def _denom_reduce(den_t, stage, accb, tmpb, den_o, cid, sid):
    pltpu.sync_copy(den_t, stage.at[sid])
    plsc.subcore_barrier()
    sl_lo = sid * R16

    def zed(i, _):
        accb[pl.ds(pl.multiple_of(i * 16, 16), 16)] = jnp.zeros((16,), f32)
        return 0

    lax.fori_loop(0, R16 // 16, zed, 0)
    for t in range(16):
        pltpu.sync_copy(stage.at[t, pl.ds(sl_lo, R16)], tmpb)

        def add(i, _):
            sl = pl.ds(pl.multiple_of(i * 16, 16), 16)
            accb[sl] = accb[sl] + tmpb[sl]
            return 0

        lax.fori_loop(0, R16 // 16, add, 0)
    pltpu.sync_copy(accb, den_o.at[cid, pl.ds(sl_lo, R16)])


def _sc_gat_a_body(s_h, d_h, as_h, ad_h, e_o, den_o,
                   as_t, ad_t, den_t, sbuf, dbuf, ebuf, accb, tmpb, stage):
    cid = lax.axis_index("c")
    sid = lax.axis_index("s")
    wid = sid * 2 + cid
    pltpu.sync_copy(as_h, as_t)
    pltpu.sync_copy(ad_h, ad_t)

    def zed(i, _):
        den_t[pl.ds(pl.multiple_of(i * 16, 16), 16)] = jnp.zeros((16,), f32)
        return 0

    lax.fori_loop(0, N8 // 16, zed, 0)
    base = wid * EPT

    def chunk(c, _):
        off = pl.multiple_of(base + c * C, 8)
        pltpu.sync_copy(s_h.at[pl.ds(off, C)], sbuf)
        pltpu.sync_copy(d_h.at[pl.ds(off, C)], dbuf)
        for j in range(C // 16):
            sl = pl.ds(j * 16, 16)
            si = sbuf[sl]
            di = dbuf[sl]
            a = plsc.load_gather(as_t, [si]) + plsc.load_gather(ad_t, [di])
            ev = jnp.exp(_leaky(a))
            ebuf[sl] = ev
            plsc.addupdate_scatter(den_t, [di], ev)
        pltpu.sync_copy(ebuf, e_o.at[pl.ds(off, C)])
        return 0

    lax.fori_loop(0, EPT // C, chunk, 0)
    _denom_reduce(den_t, stage, accb, tmpb, den_o, cid, sid)


def _sc_gat_a(s, d, asrc, adst):
    f = pl.kernel(
        _sc_gat_a_body,
        out_type=[
            jax.ShapeDtypeStruct((EA,), f32),
            jax.ShapeDtypeStruct((2, N8), f32),
        ],
        mesh=_MESH,
        compiler_params=pltpu.CompilerParams(needs_layout_passes=False),
        scratch_types=[
            pltpu.VMEM((N8,), f32),
            pltpu.VMEM((N8,), f32),
            pltpu.VMEM((N8,), f32),
            pltpu.VMEM((C,), i32),
            pltpu.VMEM((C,), i32),
            pltpu.VMEM((C,), f32),
            pltpu.VMEM((R16,), f32),
            pltpu.VMEM((R16,), f32),
            pltpu.VMEM_SHARED((16, N8), f32),
        ],
    )
    return f(s, d, asrc, adst)


def _sc_gate_a_body(s_h, d_h, ea_h, u_h, r_h, wl_h, al_h, e_o, den_o,
                    r_t, den_t, wl_t, al_t, sbuf, dbuf, eabuf, ebuf,
                    urows, accb, tmpb, stage, sem):
    cid = lax.axis_index("c")
    sid = lax.axis_index("s")
    wid = sid * 2 + cid
    pltpu.sync_copy(r_h, r_t)
    pltpu.sync_copy(wl_h, wl_t)
    pltpu.sync_copy(al_h, al_t)

    def zed(i, _):
        den_t[pl.ds(pl.multiple_of(i * 16, 16), 16)] = jnp.zeros((16,), f32)
        return 0

    lax.fori_loop(0, N8 // 16, zed, 0)
    wls = [wl_t[pl.ds(g * 16, 16)] for g in range(16)]
    als = [al_t[pl.ds(g * 16, 16)] for g in range(16)]
    base = wid * EPT

    def chunk(c, _):
        off = pl.multiple_of(base + c * C, 8)
        pltpu.sync_copy(s_h.at[pl.ds(off, C)], sbuf)
        pltpu.sync_copy(d_h.at[pl.ds(off, C)], dbuf)
        pltpu.sync_copy(ea_h.at[pl.ds(off, C)], eabuf)
        pltpu.async_copy(u_h.at[sbuf], urows, sem).wait()

        def edge(rr, _):
            ridx = jnp.full((16,), rr, i32)
            eav = plsc.load_gather(eabuf, [ridx])
            acc = jnp.zeros((16,), f32)
            for g in range(16):
                ug = urows[rr, pl.ds(g * 16, 16)]
                t = ug + eav * wls[g]
                acc = acc + als[g] * jnp.maximum(t, 0.01 * t)
            tot = jnp.sum(acc)
            dv = plsc.load_gather(dbuf, [ridx])
            rv = plsc.load_gather(r_t, [dv])
            ev = jnp.exp(_leaky(tot + rv))
            plsc.store_scatter(ebuf, [ridx], ev)
            return 0

        lax.fori_loop(0, C, edge, 0)
        for j in range(C // 16):
            sl = pl.ds(j * 16, 16)
            plsc.addupdate_scatter(den_t, [dbuf[sl]], ebuf[sl])
        pltpu.sync_copy(ebuf, e_o.at[pl.ds(off, C)])
        return 0

    lax.fori_loop(0, EPT // C, chunk, 0)
    _denom_reduce(den_t, stage, accb, tmpb, den_o, cid, sid)


def _sc_gate_a(s, d, ea, u, r, wl, al):
    f = pl.kernel(
        _sc_gate_a_body,
        out_type=[
            jax.ShapeDtypeStruct((EA,), f32),
            jax.ShapeDtypeStruct((2, N8), f32),
        ],
        mesh=_MESH,
        compiler_params=pltpu.CompilerParams(needs_layout_passes=False),
        scratch_types=[
            pltpu.VMEM((N8,), f32),
            pltpu.VMEM((N8,), f32),
            pltpu.VMEM((HID,), f32),
            pltpu.VMEM((HID,), f32),
            pltpu.VMEM((C,), i32),
            pltpu.VMEM((C,), i32),
            pltpu.VMEM((C,), f32),
            pltpu.VMEM((C,), f32),
            pltpu.VMEM((C, HID), f32),
            pltpu.VMEM((R16,), f32),
            pltpu.VMEM((R16,), f32),
            pltpu.VMEM_SHARED((16, N8), f32),
            pltpu.SemaphoreType.DMA,
        ],
    )
    return f(s, d, ea, u, r, wl, al)


def _sc_pass_b_body(s_h, d_h, e_h, den_h, v0_h, v1_h, m_o,
                    den_t, tmpn, sbuf, dbuf, ebuf, wbuf, rows, slab, sem):
    cid = lax.axis_index("c")
    sid = lax.axis_index("s")
    pltpu.sync_copy(den_h.at[0], den_t)
    pltpu.sync_copy(den_h.at[1], tmpn)

    def addn(i, _):
        sl = pl.ds(pl.multiple_of(i * 16, 16), 16)
        den_t[sl] = den_t[sl] + tmpn[sl]
        return 0

    lax.fori_loop(0, N8 // 16, addn, 0)

    def zrow(rr, _):
        for k in range(8):
            rows[rr, pl.ds(k * 16, 16)] = jnp.zeros((16,), f32)
        return 0

    lax.fori_loop(0, C, zrow, 0)
    for i in range(R16 // C):
        pltpu.sync_copy(rows, slab.at[pl.ds(sid * R16 + i * C, C)])
    plsc.subcore_barrier()

    base = sid * EPT1

    def chunk(c, _):
        off = pl.multiple_of(base + c * C, 8)
        pltpu.sync_copy(s_h.at[pl.ds(off, C)], sbuf)
        pltpu.sync_copy(d_h.at[pl.ds(off, C)], dbuf)
        pltpu.sync_copy(e_h.at[pl.ds(off, C)], ebuf)
        for j in range(C // 16):
            sl = pl.ds(j * 16, 16)
            dv = plsc.load_gather(den_t, [dbuf[sl]])
            wbuf[sl] = ebuf[sl] / (dv + 1e-16)

        @pl.when(cid == 0)
        def _():
            pltpu.async_copy(v0_h.at[sbuf], rows, sem).wait()

        @pl.when(cid == 1)
        def _():
            pltpu.async_copy(v1_h.at[sbuf], rows, sem).wait()

        def scale(rr, _):
            wv = plsc.load_gather(wbuf, [jnp.full((16,), rr, i32)])
            for k in range(8):
                sl = pl.ds(k * 16, 16)
                rows[rr, sl] = rows[rr, sl] * wv
            return 0

        lax.fori_loop(0, C, scale, 0)
        pltpu.sync_copy(rows, slab.at[dbuf], add=True)
        return 0

    lax.fori_loop(0, EPT1 // C, chunk, 0)
    plsc.subcore_barrier()
    pltpu.sync_copy(slab.at[pl.ds(sid * R16, R16)],
                    m_o.at[cid, pl.ds(sid * R16, R16)])


def _sc_pass_b(s, d, e, den, v0, v1):
    f = pl.kernel(
        _sc_pass_b_body,
        out_type=jax.ShapeDtypeStruct((2, N8, 128), f32),
        mesh=_MESH,
        compiler_params=pltpu.CompilerParams(needs_layout_passes=False),
        scratch_types=[
            pltpu.VMEM((N8,), f32),
            pltpu.VMEM((N8,), f32),
            pltpu.VMEM((C,), i32),
            pltpu.VMEM((C,), i32),
            pltpu.VMEM((C,), f32),
            pltpu.VMEM((C,), f32),
            pltpu.VMEM((C, 128), f32),
            pltpu.VMEM_SHARED((N8, 128), f32),
            pltpu.SemaphoreType.DMA,
        ],
    )
    return f(s, d, e, den, v0, v1)


def kernel(x, edge_index, edge_attr, batch, params):
    pad_e = EA - E
    s = jnp.pad(edge_index[0], (0, pad_e))
    d = jnp.pad(edge_index[1], (0, pad_e),
                constant_values=0) .at[E:].set(N + (jnp.arange(pad_e) % (N8 - N)))
    ea = jnp.pad(edge_attr[:, 0], (0, pad_e))
    xp = jnp.pad(x, ((0, N8 - N), (0, 128 - IN)))
    batch_p = jnp.pad(batch, (0, N8 - N), constant_values=B).reshape(N8, 1)

    g = params['gate']
    x1, u, v0, v1, r = _stage0(
        xp,
        jnp.pad(params['lin1_w'].T, ((0, 128 - IN), (0, 0))),
        params['lin1_b'].reshape(1, HID),
        g['lin1'][:, :HID].T,
        g['lin2'].T,
        g['att_r'].reshape(1, HID),
    )
    e1, den1 = _sc_gate_a(s, d, ea, u, r.reshape(N8),
                          g['lin1'][:, HID], g['att_l'])
    m = _sc_pass_b(s, d, e1, den1, v0, v1)
    xc, hs0, hs1, asrc, adst = _gru_prep(m, x1, g['bias'], params['gru0'],
                                         params['atom_conv'][0])
    for l in range(NUM_LAYERS - 1):
        ee, den = _sc_gat_a(s, d, asrc.reshape(N8), adst.reshape(N8))
        m = _sc_pass_b(s, d, ee, den, hs0, hs1)
        nxt = params['atom_conv'][l + 1] if l < NUM_LAYERS - 2 else params['mol_conv']
        xc, hs0, hs1, asrc, adst = _gru_prep(m, xc, params['atom_conv'][l]['bias'],
                                             params['atom_gru'][l], nxt)
    out, adst_m = _pool(xc, batch_p, params['mol_conv'])
    res = None
    for t in range(NUM_T):
        out, adst_m, res = _mol_sweep(hs0, hs1, asrc, batch_p, adst_m, out,
                                      params['mol_gru'], params['mol_conv'],
                                      params['lin2_w'], params['lin2_b'])
    return res
